# baseline (speedup 1.0000x reference)
import jax
import jax.numpy as jnp
from jax.experimental import pallas as pl
from jax.experimental.pallas import tpu as pltpu

DIM = 768
DEPTH = 2
NCLASS = 16
NSPLIT = 8


def _body(idx_ref, emb_any, Wv_any, Wo_any, W1_any, W2_any,
          pos_any, ln1g_any, ln1b_any, ln2g_any, ln2b_any,
          b1_any, b2_any, Wcls_any, bcls_any,
          out_ref, e_v, Wv_v, Wo_v, W1_v, W2_v,
          pos_v, ln1g_v, ln1b_v, ln2g_v, ln2b_v, b1_v, b2_v, Wcls_v, bcls_v,
          sems):
    B = e_v.shape[0]
    sem_i = [0]

    def copy(src, dst):
        c = pltpu.make_async_copy(src, dst, sems.at[sem_i[0]])
        sem_i[0] += 1
        c.start()
        return c

    ecp = [copy(emb_any.at[pl.ds(idx_ref[b, 0], 1), :], e_v.at[pl.ds(b, 1), :])
           for b in range(B)]
    c_pos = copy(pos_any.at[pl.ds(0, 1), :], pos_v)
    c_ln1g = copy(ln1g_any, ln1g_v)
    c_ln1b = copy(ln1b_any, ln1b_v)
    c_ln2g = copy(ln2g_any, ln2g_v)
    c_ln2b = copy(ln2b_any, ln2b_v)
    c_b1 = copy(b1_any, b1_v)
    c_b2 = copy(b2_any, b2_v)
    c_wcls = copy(Wcls_any, Wcls_v)
    c_bcls = copy(bcls_any, bcls_v)

    def chunked(src, dst, l):
        rows = src.shape[1]
        cs = []
        for c in range(NSPLIT):
            r0, r1 = c * rows // NSPLIT, (c + 1) * rows // NSPLIT
            cs.append(copy(src.at[l, pl.ds(r0, r1 - r0), :],
                           dst.at[l, pl.ds(r0, r1 - r0), :]))
        return cs

    wcp = [[chunked(Wv_any, Wv_v, l), chunked(Wo_any, Wo_v, l),
            chunked(W1_any, W1_v, l), chunked(W2_any, W2_v, l)]
           for l in range(DEPTH)]

    def ln(v, g, bb):
        mu = jnp.mean(v, axis=-1, keepdims=True)
        var = jnp.mean((v - mu) ** 2, axis=-1, keepdims=True)
        return (v - mu) / jnp.sqrt(var + 1e-5) * g + bb

    for c in ecp:
        c.wait()
    c_pos.wait()
    h = e_v[...] + pos_v[...]

    c_ln1g.wait()
    c_ln1b.wait()
    c_ln2g.wait()
    c_ln2b.wait()
    c_b1.wait()
    c_b2.wait()
    for l in range(DEPTH):
        sl = pl.ds(l, 1)
        xln = ln(h, ln1g_v[sl, :], ln1b_v[sl, :])
        for c in wcp[l][0]:
            c.wait()
        a = jnp.dot(xln, Wv_v[l], preferred_element_type=jnp.float32)
        for c in wcp[l][1]:
            c.wait()
        a = jnp.dot(a, Wo_v[l], preferred_element_type=jnp.float32)
        h = h + a
        fln = ln(h, ln2g_v[sl, :], ln2b_v[sl, :])
        for c in wcp[l][2]:
            c.wait()
        f = jax.nn.gelu(jnp.dot(fln, W1_v[l], preferred_element_type=jnp.float32)
                        + b1_v[sl, :])
        acc = b2_v[sl, :]
        rows = W2_v.shape[1]
        for c in range(NSPLIT):
            r0, r1 = c * rows // NSPLIT, (c + 1) * rows // NSPLIT
            wcp[l][3][c].wait()
            acc = acc + jnp.dot(f[:, r0:r1], W2_v[l, r0:r1, :],
                                preferred_element_type=jnp.float32)
        h = h + acc

    c_wcls.wait()
    c_bcls.wait()
    out_ref[...] = jnp.dot(h, Wcls_v[...],
                           preferred_element_type=jnp.float32) + bcls_v[...]


def kernel(x, emb, pos, ln1_g, ln1_b, Wqk, Wv, Wo, ln2_g, ln2_b,
           W1, b1, W2, b2, rot, Wcls, bcls):
    B = x.shape[0]

    any_spec = pl.BlockSpec(memory_space=pl.MemorySpace.ANY)
    v2 = lambda r, c: pltpu.VMEM((r, c), jnp.float32)

    out = pl.pallas_call(
        _body,
        grid_spec=pltpu.PrefetchScalarGridSpec(
            num_scalar_prefetch=1,
            grid=(1,),
            in_specs=[any_spec] * 14,
            out_specs=pl.BlockSpec((B, NCLASS), lambda i, idx: (0, 0)),
            scratch_shapes=[
                v2(B, DIM),
                pltpu.VMEM((DEPTH, DIM, DIM), jnp.float32),
                pltpu.VMEM((DEPTH, DIM, DIM), jnp.float32),
                pltpu.VMEM((DEPTH, DIM, 4 * DIM), jnp.float32),
                pltpu.VMEM((DEPTH, 4 * DIM, DIM), jnp.float32),
                v2(1, DIM),
                v2(DEPTH, DIM), v2(DEPTH, DIM),
                v2(DEPTH, DIM), v2(DEPTH, DIM),
                v2(DEPTH, 4 * DIM), v2(DEPTH, DIM),
                v2(DIM, NCLASS), v2(1, NCLASS),
                pltpu.SemaphoreType.DMA((B + 10 + 4 * DEPTH * NSPLIT,)),
            ],
        ),
        out_shape=jax.ShapeDtypeStruct((B, NCLASS), jnp.float32),
    )(x, emb, Wv, Wo, W1, W2, pos, ln1_g, ln1_b, ln2_g, ln2_b,
      b1, b2, Wcls, bcls.reshape(1, NCLASS))
    return out


# speedup vs baseline: 1.0287x; 1.0287x over previous
import jax
import jax.numpy as jnp
from jax.experimental import pallas as pl
from jax.experimental.pallas import tpu as pltpu

DIM = 768
DEPTH = 2
NCLASS = 16
NSPLIT = 2


def _body(idx_ref, emb_any, Wv_any, Wo_any, W1_any, W2_any,
          pos_any, ln1g_any, ln1b_any, ln2g_any, ln2b_any,
          b1_any, b2_any, Wcls_any, bcls_any,
          out_ref, e_v, Wv_v, Wo_v, W1_v, W2_v,
          pos_v, ln1g_v, ln1b_v, ln2g_v, ln2b_v, b1_v, b2_v, Wcls_v, bcls_v,
          sems):
    B = e_v.shape[0]
    sem_i = [0]

    def copy(src, dst):
        c = pltpu.make_async_copy(src, dst, sems.at[sem_i[0]])
        sem_i[0] += 1
        c.start()
        return c

    ecp = [copy(emb_any.at[pl.ds(idx_ref[b, 0], 1), :], e_v.at[pl.ds(b, 1), :])
           for b in range(B)]
    c_pos = copy(pos_any.at[pl.ds(0, 1), :], pos_v)
    c_ln1g = copy(ln1g_any, ln1g_v)
    c_ln1b = copy(ln1b_any, ln1b_v)
    c_ln2g = copy(ln2g_any, ln2g_v)
    c_ln2b = copy(ln2b_any, ln2b_v)
    c_b1 = copy(b1_any, b1_v)
    c_b2 = copy(b2_any, b2_v)
    c_wcls = copy(Wcls_any, Wcls_v)
    c_bcls = copy(bcls_any, bcls_v)

    def chunked(src, dst, l):
        rows = src.shape[1]
        cs = []
        for c in range(NSPLIT):
            r0, r1 = c * rows // NSPLIT, (c + 1) * rows // NSPLIT
            cs.append(copy(src.at[l, pl.ds(r0, r1 - r0), :],
                           dst.at[l, pl.ds(r0, r1 - r0), :]))
        return cs

    wcp = [[chunked(Wv_any, Wv_v, l), chunked(Wo_any, Wo_v, l),
            chunked(W1_any, W1_v, l), chunked(W2_any, W2_v, l)]
           for l in range(DEPTH)]

    def ln(v, g, bb):
        mu = jnp.mean(v, axis=-1, keepdims=True)
        var = jnp.mean((v - mu) ** 2, axis=-1, keepdims=True)
        return (v - mu) / jnp.sqrt(var + 1e-5) * g + bb

    for c in ecp:
        c.wait()
    c_pos.wait()
    h = e_v[...] + pos_v[...]

    c_ln1g.wait()
    c_ln1b.wait()
    c_ln2g.wait()
    c_ln2b.wait()
    c_b1.wait()
    c_b2.wait()
    for l in range(DEPTH):
        sl = pl.ds(l, 1)
        xln = ln(h, ln1g_v[sl, :], ln1b_v[sl, :])
        for c in wcp[l][0]:
            c.wait()
        a = jnp.dot(xln, Wv_v[l], preferred_element_type=jnp.float32)
        for c in wcp[l][1]:
            c.wait()
        a = jnp.dot(a, Wo_v[l], preferred_element_type=jnp.float32)
        h = h + a
        fln = ln(h, ln2g_v[sl, :], ln2b_v[sl, :])
        for c in wcp[l][2]:
            c.wait()
        f = jax.nn.gelu(jnp.dot(fln, W1_v[l], preferred_element_type=jnp.float32)
                        + b1_v[sl, :])
        acc = b2_v[sl, :]
        rows = W2_v.shape[1]
        for c in range(NSPLIT):
            r0, r1 = c * rows // NSPLIT, (c + 1) * rows // NSPLIT
            wcp[l][3][c].wait()
            acc = acc + jnp.dot(f[:, r0:r1], W2_v[l, r0:r1, :],
                                preferred_element_type=jnp.float32)
        h = h + acc

    c_wcls.wait()
    c_bcls.wait()
    out_ref[...] = jnp.dot(h, Wcls_v[...],
                           preferred_element_type=jnp.float32) + bcls_v[...]


def kernel(x, emb, pos, ln1_g, ln1_b, Wqk, Wv, Wo, ln2_g, ln2_b,
           W1, b1, W2, b2, rot, Wcls, bcls):
    B = x.shape[0]

    any_spec = pl.BlockSpec(memory_space=pl.MemorySpace.ANY)
    v2 = lambda r, c: pltpu.VMEM((r, c), jnp.float32)

    out = pl.pallas_call(
        _body,
        grid_spec=pltpu.PrefetchScalarGridSpec(
            num_scalar_prefetch=1,
            grid=(1,),
            in_specs=[any_spec] * 14,
            out_specs=pl.BlockSpec((B, NCLASS), lambda i, idx: (0, 0)),
            scratch_shapes=[
                v2(B, DIM),
                pltpu.VMEM((DEPTH, DIM, DIM), jnp.float32),
                pltpu.VMEM((DEPTH, DIM, DIM), jnp.float32),
                pltpu.VMEM((DEPTH, DIM, 4 * DIM), jnp.float32),
                pltpu.VMEM((DEPTH, 4 * DIM, DIM), jnp.float32),
                v2(1, DIM),
                v2(DEPTH, DIM), v2(DEPTH, DIM),
                v2(DEPTH, DIM), v2(DEPTH, DIM),
                v2(DEPTH, 4 * DIM), v2(DEPTH, DIM),
                v2(DIM, NCLASS), v2(1, NCLASS),
                pltpu.SemaphoreType.DMA((B + 10 + 4 * DEPTH * NSPLIT,)),
            ],
        ),
        out_shape=jax.ShapeDtypeStruct((B, NCLASS), jnp.float32),
    )(x, emb, Wv, Wo, W1, W2, pos, ln1_g, ln1_b, ln2_g, ln2_b,
      b1, b2, Wcls, bcls.reshape(1, NCLASS))
    return out
